# baseline (speedup 1.0000x reference)
import functools
import statistics

import numpy as np
import jax
import jax.numpy as jnp
from jax import lax
from jax.experimental import pallas as pl
from jax.experimental.pallas import tpu as pltpu
from jax.experimental.pallas import tpu_sc as plsc

_NUM_EMB = 1000000
_D = 64
_B = 16384
_F = 26
_N = _B * _F

_TBR = 7936
_POFF = 63 * _TBR
_P = _NUM_EMB - _POFF


def _nf5():
    nd = statistics.NormalDist()
    offset = 0.9677083
    pos = [nd.inv_cdf(p) for p in np.linspace(offset, 0.5, 17)[:-1]]
    neg = [-nd.inv_cdf(p) for p in np.linspace(offset, 0.5, 16)[:-1]]
    vals = np.array(sorted(pos + [0.0] + neg), dtype=np.float32)
    return vals / np.abs(vals).max()


_LV = _nf5()
_MID = ((_LV[:-1] + _LV[1:]) * np.float32(0.5)).astype(np.float32)
_LVI = np.round(_LV * 32767.0).astype(np.int16)
_MIDI = np.round(_MID * 16384.0).astype(np.int16)


def _tp_body(a_ref, b_ref, o_ref):
    o_ref[:, :_D] = a_ref[...].T
    o_ref[:, _D:] = b_ref[...].T


_transpose = pl.pallas_call(
    _tp_body,
    grid=(64,),
    in_specs=[
        pl.BlockSpec((_D, _TBR), lambda g: (0, g)),
        pl.BlockSpec((_D, _TBR), lambda g: (0, g + 63)),
    ],
    out_specs=pl.BlockSpec((_TBR, 128), lambda g: (g, 0)),
    out_shape=jax.ShapeDtypeStruct((_P, 128), jnp.float32),
)


_NC, _NS = 2, 16
_NW = _NC * _NS
_ROWS_PER_W = _N // _NW
_CHUNK = 512
_NCHUNK = _ROWS_PER_W // _CHUNK
_IPG = 128
_GPC = _CHUNK // _IPG
_NSLAB = _N // 2048


@functools.cache
def _make_gather():
    mesh = plsc.VectorSubcoreMesh(core_axis_name="c", subcore_axis_name="s")

    @functools.partial(
        pl.kernel,
        mesh=mesh,
        compiler_params=pltpu.CompilerParams(use_tc_tiling_on_sc=False),
        out_type=jax.ShapeDtypeStruct((_NSLAB * 1024, 256), jnp.float32),
        scratch_types=[
            pltpu.VMEM((_GPC, _IPG), jnp.int32),
            pltpu.VMEM((_CHUNK, 128), jnp.float32),
            pltpu.SemaphoreType.DMA,
        ],
    )
    def gather(idx2_hbm, table_hbm, out_hbm, idx_v, rows_v, sem):
        wid = lax.axis_index("s") * _NC + lax.axis_index("c")
        cbase = wid * _NCHUNK

        def body(i, carry):
            c = cbase + i
            irow = pl.multiple_of(c * _GPC, _GPC)
            pltpu.sync_copy(idx2_hbm.at[pl.ds(irow, _GPC)], idx_v)
            copies = []
            for j in range(_GPC):
                copies.append(
                    pltpu.async_copy(
                        table_hbm.at[idx_v.at[j]],
                        rows_v.at[pl.ds(j * _IPG, _IPG)],
                        sem,
                    )
                )
            for cp in copies:
                cp.wait()
            ro = pl.multiple_of((c // 4) * 1024 + (c % 2) * _CHUNK, _CHUNK)
            lo = pl.multiple_of(((c // 2) % 2) * 128, 128)
            pltpu.sync_copy(
                rows_v, out_hbm.at[pl.ds(ro, _CHUNK), pl.ds(lo, 128)]
            )
            return carry

        lax.fori_loop(0, _NCHUNK, body, 0)

    return gather


_BB = 4096
_JB = _B // _BB


def _fq(y):
    amax = jnp.max(jnp.abs(y), axis=0, keepdims=True)
    scale = jnp.where(amax > 0.0, amax, 1.0)
    si = (y * (16384.0 / scale)).astype(jnp.int16)
    q = jnp.full(y.shape, _LVI[0], jnp.int16)
    for k in range(1, 32):
        q = jnp.where(si > _MIDI[k - 1], jnp.int16(_LVI[k]), q)
    return q.astype(jnp.float32) * (scale * (1.0 / 32767.0))


def _quant_body(x_ref, h_ref, o_ref):
    x = x_ref[...]
    xt = x.T
    hv = h_ref[...].reshape(1, _BB)
    for s in range(2):
        for qq in range(2):
            lo = s * 2048 + qq * 1024
            hseg = hv[:, lo : lo + 1024] != 0
            left = xt[qq * 128 : qq * 128 + _D, s * 1024 : (s + 1) * 1024]
            right = xt[qq * 128 + _D : qq * 128 + 128, s * 1024 : (s + 1) * 1024]
            y = jnp.where(hseg, right, left)
            o_ref[0, :, lo : lo + 1024] = _fq(y)


_quantize = pl.pallas_call(
    _quant_body,
    grid=(_F, _JB),
    in_specs=[
        pl.BlockSpec((2048, 256), lambda f, j: (f * _JB + j, 0)),
        pl.BlockSpec((1, 1, _BB), lambda f, j: (f * _JB + j, 0, 0)),
    ],
    out_specs=pl.BlockSpec((1, _D, _BB), lambda f, j: (f, 0, j)),
    out_shape=jax.ShapeDtypeStruct((_F, _D, _B), jnp.float32),
)


def kernel(input, weight):
    iT = input.T.astype(jnp.int32)
    h = (iT >= _P).astype(jnp.int32)
    p = iT - _POFF * h
    idx2 = p.reshape(_N // _IPG, _IPG)
    h3 = h.reshape(_F * _JB, 1, _BB)
    table2 = _transpose(weight.T, weight.T)
    gathered = _make_gather()(idx2, table2)
    q = _quantize(gathered, h3)
    return q.transpose(2, 0, 1)


# speedup vs baseline: 1.3634x; 1.3634x over previous
import functools
import statistics

import numpy as np
import jax
import jax.numpy as jnp
from jax import lax
from jax.experimental import pallas as pl
from jax.experimental.pallas import tpu as pltpu
from jax.experimental.pallas import tpu_sc as plsc

_NUM_EMB = 1000000
_D = 64
_B = 16384
_F = 26
_N = _B * _F

_TBR = 7936
_POFF = 63 * _TBR
_P = _NUM_EMB - _POFF


def _nf5():
    nd = statistics.NormalDist()
    offset = 0.9677083
    pos = [nd.inv_cdf(p) for p in np.linspace(offset, 0.5, 17)[:-1]]
    neg = [-nd.inv_cdf(p) for p in np.linspace(offset, 0.5, 16)[:-1]]
    vals = np.array(sorted(pos + [0.0] + neg), dtype=np.float32)
    return vals / np.abs(vals).max()


_LV = _nf5()
_MID = ((_LV[:-1] + _LV[1:]) * np.float32(0.5)).astype(np.float32)
_LVI = np.round(_LV * 32767.0).astype(np.int16)
_MIDI = np.round(_MID * 16384.0).astype(np.int16)


def _tp_body(a_ref, b_ref, o_ref):
    o_ref[:, :_D] = a_ref[...].T
    o_ref[:, _D:] = b_ref[...].T


_transpose = pl.pallas_call(
    _tp_body,
    grid=(64,),
    in_specs=[
        pl.BlockSpec((_D, _TBR), lambda g: (0, g)),
        pl.BlockSpec((_D, _TBR), lambda g: (0, g + 63)),
    ],
    out_specs=pl.BlockSpec((_TBR, 128), lambda g: (g, 0)),
    out_shape=jax.ShapeDtypeStruct((_P, 128), jnp.float32),
)


_NC, _NS = 2, 16
_NW = _NC * _NS
_ROWS_PER_W = _N // _NW
_CHUNK = 512
_NCHUNK = _ROWS_PER_W // _CHUNK
_IPG = 128
_GPC = _CHUNK // _IPG
_NSLAB = _N // 2048


@functools.cache
def _make_gather():
    mesh = plsc.VectorSubcoreMesh(core_axis_name="c", subcore_axis_name="s")

    @functools.partial(
        pl.kernel,
        mesh=mesh,
        compiler_params=pltpu.CompilerParams(use_tc_tiling_on_sc=False),
        out_type=jax.ShapeDtypeStruct((_N, 128), jnp.float32),
        scratch_types=[
            pltpu.VMEM((_GPC, _IPG), jnp.int32),
            pltpu.VMEM((_CHUNK, 128), jnp.float32),
            pltpu.SemaphoreType.DMA,
        ],
    )
    def gather(idx2_hbm, table_hbm, out_hbm, idx_v, rows_v, sem):
        wid = lax.axis_index("s") * _NC + lax.axis_index("c")
        cbase = wid * _NCHUNK

        def body(i, carry):
            c = cbase + i
            irow = pl.multiple_of(c * _GPC, _GPC)
            pltpu.sync_copy(idx2_hbm.at[pl.ds(irow, _GPC)], idx_v)
            copies = []
            for j in range(_GPC):
                copies.append(
                    pltpu.async_copy(
                        table_hbm.at[idx_v.at[j]],
                        rows_v.at[pl.ds(j * _IPG, _IPG)],
                        sem,
                    )
                )
            for cp in copies:
                cp.wait()
            ro = pl.multiple_of(c * _CHUNK, _CHUNK)
            pltpu.sync_copy(rows_v, out_hbm.at[pl.ds(ro, _CHUNK)])
            return carry

        lax.fori_loop(0, _NCHUNK, body, 0)

    return gather


_BB = 4096
_JB = _B // _BB


def _fq(y):
    amax = jnp.max(jnp.abs(y), axis=0, keepdims=True)
    scale = jnp.where(amax > 0.0, amax, 1.0)
    si = (y * (16384.0 / scale)).astype(jnp.int16)
    q = jnp.full(y.shape, _LVI[0], jnp.int16)
    for k in range(1, 32):
        q = jnp.where(si > _MIDI[k - 1], jnp.int16(_LVI[k]), q)
    return q.astype(jnp.float32) * (scale * (1.0 / 32767.0))


def _quant_body(x_ref, h_ref, o_ref):
    x = x_ref[...]
    xt = x.T
    hv = h_ref[...].reshape(1, _BB) != 0
    y = jnp.where(hv, xt[_D:], xt[:_D])
    o_ref[0] = _fq(y)


_quantize = pl.pallas_call(
    _quant_body,
    grid=(_F, _JB),
    in_specs=[
        pl.BlockSpec((_BB, 128), lambda f, j: (f * _JB + j, 0)),
        pl.BlockSpec((1, 1, _BB), lambda f, j: (f * _JB + j, 0, 0)),
    ],
    out_specs=pl.BlockSpec((1, _D, _BB), lambda f, j: (f, 0, j)),
    out_shape=jax.ShapeDtypeStruct((_F, _D, _B), jnp.float32),
)


def kernel(input, weight):
    iT = input.T.astype(jnp.int32)
    h = (iT >= _P).astype(jnp.int32)
    p = iT - _POFF * h
    idx2 = p.reshape(_N // _IPG, _IPG)
    h3 = h.reshape(_F * _JB, 1, _BB)
    table2 = _transpose(weight.T, weight.T)
    gathered = _make_gather()(idx2, table2)
    q = _quantize(gathered, h3)
    return q.transpose(2, 0, 1)


# speedup vs baseline: 1.3758x; 1.0091x over previous
import functools
import statistics

import numpy as np
import jax
import jax.numpy as jnp
from jax import lax
from jax.experimental import pallas as pl
from jax.experimental.pallas import tpu as pltpu
from jax.experimental.pallas import tpu_sc as plsc

_NUM_EMB = 1000000
_D = 64
_B = 16384
_F = 26
_N = _B * _F

_TBR = 7936
_POFF = 63 * _TBR
_P = _NUM_EMB - _POFF


def _nf5():
    nd = statistics.NormalDist()
    offset = 0.9677083
    pos = [nd.inv_cdf(p) for p in np.linspace(offset, 0.5, 17)[:-1]]
    neg = [-nd.inv_cdf(p) for p in np.linspace(offset, 0.5, 16)[:-1]]
    vals = np.array(sorted(pos + [0.0] + neg), dtype=np.float32)
    return vals / np.abs(vals).max()


_LV = _nf5()
_MID = ((_LV[:-1] + _LV[1:]) * np.float32(0.5)).astype(np.float32)
_LVI = np.round(_LV * 32767.0).astype(np.int16)
_MIDI = np.round(_MID * 16384.0).astype(np.int16)


def _tp_body(a_ref, b_ref, o_ref):
    o_ref[:, :_D] = a_ref[...].T
    o_ref[:, _D:] = b_ref[...].T


_transpose = pl.pallas_call(
    _tp_body,
    grid=(64,),
    in_specs=[
        pl.BlockSpec((_D, _TBR), lambda g: (0, g)),
        pl.BlockSpec((_D, _TBR), lambda g: (0, g + 63)),
    ],
    out_specs=pl.BlockSpec((_TBR, 128), lambda g: (g, 0)),
    out_shape=jax.ShapeDtypeStruct((_P, 128), jnp.float32),
)


_NC, _NS = 2, 16
_NW = _NC * _NS
_ROWS_PER_W = _N // _NW
_CHUNK = 512
_NCHUNK = _ROWS_PER_W // _CHUNK
_IPG = 128
_GPC = _CHUNK // _IPG
_NSLAB = _N // 2048


@functools.cache
def _make_gather():
    mesh = plsc.VectorSubcoreMesh(core_axis_name="c", subcore_axis_name="s")

    @functools.partial(
        pl.kernel,
        mesh=mesh,
        compiler_params=pltpu.CompilerParams(use_tc_tiling_on_sc=False),
        out_type=jax.ShapeDtypeStruct((_N, 128), jnp.float32),
        scratch_types=[
            pltpu.VMEM((_GPC, _IPG), jnp.int32),
            pltpu.VMEM((_CHUNK, 128), jnp.float32),
            pltpu.SemaphoreType.DMA,
        ],
    )
    def gather(idx2_hbm, table_hbm, out_hbm, idx_v, rows_v, sem):
        wid = lax.axis_index("s") * _NC + lax.axis_index("c")
        cbase = wid * _NCHUNK

        def body(i, carry):
            c = cbase + i
            irow = pl.multiple_of(c * _GPC, _GPC)
            pltpu.sync_copy(idx2_hbm.at[pl.ds(irow, _GPC)], idx_v)
            copies = []
            for j in range(_GPC):
                copies.append(
                    pltpu.async_copy(
                        table_hbm.at[idx_v.at[j]],
                        rows_v.at[pl.ds(j * _IPG, _IPG)],
                        sem,
                    )
                )
            for cp in copies:
                cp.wait()
            ro = pl.multiple_of(c * _CHUNK, _CHUNK)
            pltpu.sync_copy(rows_v, out_hbm.at[pl.ds(ro, _CHUNK)])
            return carry

        lax.fori_loop(0, _NCHUNK, body, 0)

    return gather


_BB = 8192
_JB = _B // _BB


def _fq(y):
    amax = jnp.max(jnp.abs(y), axis=0, keepdims=True)
    scale = jnp.where(amax > 0.0, amax, 1.0)
    si = (y * (16384.0 / scale)).astype(jnp.int16)
    q = jnp.full(y.shape, _LVI[0], jnp.int16)
    for k in range(1, 32):
        q = jnp.where(si > _MIDI[k - 1], jnp.int16(_LVI[k]), q)
    return q.astype(jnp.float32) * (scale * (1.0 / 32767.0))


def _quant_body(x_ref, h_ref, o_ref):
    x = x_ref[...]
    xt = x.T
    hv = h_ref[...].reshape(1, _BB) != 0
    y = jnp.where(hv, xt[_D:], xt[:_D])
    o_ref[0] = _fq(y)


_quantize = pl.pallas_call(
    _quant_body,
    grid=(_F, _JB),
    in_specs=[
        pl.BlockSpec((_BB, 128), lambda f, j: (f * _JB + j, 0)),
        pl.BlockSpec((1, 1, _BB), lambda f, j: (f * _JB + j, 0, 0)),
    ],
    out_specs=pl.BlockSpec((1, _D, _BB), lambda f, j: (f, 0, j)),
    out_shape=jax.ShapeDtypeStruct((_F, _D, _B), jnp.float32),
)


def kernel(input, weight):
    iT = input.T.astype(jnp.int32)
    h = (iT >= _P).astype(jnp.int32)
    p = iT - _POFF * h
    idx2 = p.reshape(_N // _IPG, _IPG)
    h3 = h.reshape(_F * _JB, 1, _BB)
    table2 = _transpose(weight.T, weight.T)
    gathered = _make_gather()(idx2, table2)
    q = _quantize(gathered, h3)
    return q.transpose(2, 0, 1)
